# baseline (speedup 1.0000x reference)
import functools

import jax
import jax.numpy as jnp
import numpy as np
from jax import lax
from jax.experimental import pallas as pl
from jax.experimental.pallas import tpu as pltpu
from jax.experimental.pallas import tpu_sc as plsc

N = 10000
E = 320000
D = 128
H = 128
OUT = 256
L = 5
B = 64
BN_EPS = 1e-5

NC = 2
NS = 16
NW = NC * NS
CHUNK = 128
NBUF = 2
SCH = 8
EPW = E // NW
NSUP = -(-EPW // (CHUNK * SCH))
EPW_CH = NSUP * SCH
EPW_PAD = EPW_CH * CHUNK - EPW
NPAD = 10240
RPS = NPAD // NS

R = 1000
G = N // R


def _sc_scatter_add(h, src3, dst3, zeros_blk):
    mesh = plsc.VectorSubcoreMesh(core_axis_name="c", subcore_axis_name="s")

    @functools.partial(
        pl.kernel,
        mesh=mesh,
        out_type=jax.ShapeDtypeStruct((NC, NPAD, D), jnp.float32),
        scratch_types=[
            pltpu.VMEM((NBUF, SCH, CHUNK), jnp.int32),
            pltpu.VMEM((NBUF, SCH, CHUNK), jnp.int32),
            pltpu.VMEM((NBUF * CHUNK, D), jnp.float32),
            pltpu.VMEM_SHARED((NPAD, D), jnp.float32),
            pltpu.SemaphoreType.DMA,
            pltpu.SemaphoreType.DMA,
            pltpu.SemaphoreType.DMA,
            pltpu.SemaphoreType.DMA,
            pltpu.SemaphoreType.DMA,
        ],
    )
    def k(
        h_hbm, src_hbm, dst_hbm, z_hbm, out_hbm,
        sidx, didx, rows_v, agg_sh, sg0, sg1, ss0, ss1, si,
    ):
        c = lax.axis_index("c")
        s = lax.axis_index("s")
        wid = s * NC + c
        rows = [rows_v.at[pl.ds(b * CHUNK, CHUNK)] for b in range(NBUF)]
        sg = [sg0, sg1]
        ss = [ss0, ss1]

        pltpu.sync_copy(z_hbm, rows[0])

        def zc(q, carry):
            pltpu.sync_copy(rows[0], agg_sh.at[pl.ds(s * RPS + q * CHUNK, CHUNK)])
            return carry

        lax.fori_loop(0, RPS // CHUNK, zc, 0)
        plsc.subcore_barrier()

        def drain_rows(sem, buf):
            pltpu.make_async_copy(z_hbm, buf, sem).wait()

        def drain_idx():
            pltpu.make_async_copy(src_hbm.at[wid, 0], sidx.at[0], si).wait()
            pltpu.make_async_copy(src_hbm.at[wid, 0], didx.at[0], si).wait()

        pltpu.sync_copy(src_hbm.at[wid, 0], sidx.at[0])
        pltpu.sync_copy(dst_hbm.at[wid, 0], didx.at[0])
        pltpu.async_copy(src_hbm.at[wid, 1], sidx.at[1], si)
        pltpu.async_copy(dst_hbm.at[wid, 1], didx.at[1], si)

        for b in range(NBUF):
            pltpu.async_copy(h_hbm.at[sidx.at[0, b]], rows[b], sg[b])

        def super_chunk(u, carry):
            p = lax.rem(u, 2)
            pn = 1 - p
            for q in range(SCH):
                b = q % NBUF
                drain_rows(sg[b], rows[b])
                pltpu.async_copy(rows[b], agg_sh.at[didx.at[p, q]], ss[b], add=True)
                drain_rows(ss[b], rows[b])
                if q + NBUF < SCH:
                    pltpu.async_copy(h_hbm.at[sidx.at[p, q + NBUF]], rows[b], sg[b])
                else:
                    if q + NBUF == SCH:
                        drain_idx()
                    pltpu.async_copy(
                        h_hbm.at[sidx.at[pn, q + NBUF - SCH]], rows[b], sg[b]
                    )

            un = jnp.minimum(u + 2, NSUP - 1)
            pltpu.async_copy(src_hbm.at[wid, un], sidx.at[p], si)
            pltpu.async_copy(dst_hbm.at[wid, un], didx.at[p], si)
            return carry

        lax.fori_loop(0, NSUP, super_chunk, 0)
        for b in range(NBUF):
            drain_rows(sg[b], rows[b])
        drain_idx()
        plsc.subcore_barrier()

        pltpu.sync_copy(
            agg_sh.at[pl.ds(s * RPS, RPS)], out_hbm.at[c, pl.ds(s * RPS, RPS)]
        )

    return k(h, src3, dst3, zeros_blk)


def _mlp_mid(h, aggs, W1, b1, W2, b2):
    def body(h_ref, a_ref, w1_ref, b1_ref, w2_ref, b2_ref, out_ref):
        z = h_ref[...] + a_ref[0] + a_ref[1]
        z = jnp.dot(z, w1_ref[...], preferred_element_type=jnp.float32) + b1_ref[...]
        z = jnp.maximum(z, 0.0)
        z = jnp.dot(z, w2_ref[...], preferred_element_type=jnp.float32) + b2_ref[...]
        z = jnp.maximum(z, 0.0)
        out_ref[...] = h_ref[...] + z

    return pl.pallas_call(
        body,
        grid=(G,),
        in_specs=[
            pl.BlockSpec((R, D), lambda i: (i, 0)),
            pl.BlockSpec((NC, R, D), lambda i: (0, i, 0)),
            pl.BlockSpec((D, H), lambda i: (0, 0)),
            pl.BlockSpec((1, H), lambda i: (0, 0)),
            pl.BlockSpec((H, H), lambda i: (0, 0)),
            pl.BlockSpec((1, H), lambda i: (0, 0)),
        ],
        out_specs=pl.BlockSpec((R, D), lambda i: (i, 0)),
        out_shape=jax.ShapeDtypeStruct((N, D), jnp.float32),
    )(h, aggs, W1, b1, W2, b2)


def _mlp_last(h, aggs, W1, b1, W2, b2, batch2, Wout, bout):
    def body(
        h_ref, a_ref, w1_ref, b1_ref, w2_ref, b2_ref, bt_ref, wo_ref, bo_ref,
        out_ref, g_ref, sums_ref, cnts_ref,
    ):
        i = pl.program_id(0)
        z = h_ref[...] + a_ref[0] + a_ref[1]
        z = jnp.dot(z, w1_ref[...], preferred_element_type=jnp.float32) + b1_ref[...]
        z = jnp.maximum(z, 0.0)
        z = jnp.dot(z, w2_ref[...], preferred_element_type=jnp.float32) + b2_ref[...]
        z = jnp.maximum(z, 0.0)
        hnew = h_ref[...] + z
        out_ref[...] = hnew

        onehot = (
            bt_ref[...] == lax.broadcasted_iota(jnp.int32, (R, B), 1)
        ).astype(jnp.float32)
        part = lax.dot_general(
            onehot, hnew, (((0,), (0,)), ((), ())),
            preferred_element_type=jnp.float32,
        )
        cnt = lax.dot_general(
            onehot, jnp.ones((R, 1), jnp.float32), (((0,), (0,)), ((), ())),
            preferred_element_type=jnp.float32,
        )

        @pl.when(i == 0)
        def _():
            sums_ref[...] = part
            cnts_ref[...] = cnt

        @pl.when(i > 0)
        def _():
            sums_ref[...] += part
            cnts_ref[...] += cnt

        @pl.when(i == G - 1)
        def _():
            mean = sums_ref[...] / jnp.maximum(cnts_ref[...], 1.0)
            g_ref[...] = (
                jnp.dot(mean, wo_ref[...], preferred_element_type=jnp.float32)
                + bo_ref[...]
            )

    return pl.pallas_call(
        body,
        grid=(G,),
        in_specs=[
            pl.BlockSpec((R, D), lambda i: (i, 0)),
            pl.BlockSpec((NC, R, D), lambda i: (0, i, 0)),
            pl.BlockSpec((D, H), lambda i: (0, 0)),
            pl.BlockSpec((1, H), lambda i: (0, 0)),
            pl.BlockSpec((H, H), lambda i: (0, 0)),
            pl.BlockSpec((1, H), lambda i: (0, 0)),
            pl.BlockSpec((R, 1), lambda i: (i, 0)),
            pl.BlockSpec((H, OUT), lambda i: (0, 0)),
            pl.BlockSpec((1, OUT), lambda i: (0, 0)),
        ],
        out_specs=[
            pl.BlockSpec((R, D), lambda i: (i, 0)),
            pl.BlockSpec((B, OUT), lambda i: (0, 0)),
        ],
        out_shape=[
            jax.ShapeDtypeStruct((N, D), jnp.float32),
            jax.ShapeDtypeStruct((B, OUT), jnp.float32),
        ],
        scratch_shapes=[
            pltpu.VMEM((B, H), jnp.float32),
            pltpu.VMEM((B, 1), jnp.float32),
        ],
    )(h, aggs, W1, b1, W2, b2, batch2, Wout, bout)


def kernel(x, edge_index, batch, params):
    inv = np.float32(1.0 / np.sqrt(1.0 + BN_EPS))

    src = edge_index[0].reshape(NW, EPW)
    dst = edge_index[1].reshape(NW, EPW)
    src3 = jnp.concatenate(
        [src, jnp.zeros((NW, EPW_PAD), jnp.int32)], axis=1
    ).reshape(NW, NSUP, SCH, CHUNK)
    dpad = N + (jnp.arange(EPW_PAD, dtype=jnp.int32) % (NPAD - N))
    dst3 = jnp.concatenate(
        [dst, jnp.broadcast_to(dpad[None, :], (NW, EPW_PAD))], axis=1
    ).reshape(NW, NSUP, SCH, CHUNK)
    zeros_blk = jnp.zeros((CHUNK, D), jnp.float32)
    batch2 = batch.reshape(N, 1)

    Ws1, bs1, Ws2, bs2 = [], [], [], []
    for i in range(L):
        g1 = params["l%d_g1" % i] * inv
        Ws1.append(params["l%d_W1" % i] * g1[None, :])
        bs1.append((params["l%d_b1" % i] * g1 + params["l%d_be1" % i]).reshape(1, H))
        g2 = params["l%d_bng" % i] * inv
        Ws2.append(params["l%d_W2" % i] * g2[None, :])
        bs2.append((params["l%d_b2" % i] * g2 + params["l%d_bnb" % i]).reshape(1, H))

    h = x
    for i in range(L):
        aggs = _sc_scatter_add(h, src3, dst3, zeros_blk)
        if i < L - 1:
            h = _mlp_mid(h, aggs, Ws1[i], bs1[i], Ws2[i], bs2[i])
        else:
            h, graph = _mlp_last(
                h, aggs, Ws1[i], bs1[i], Ws2[i], bs2[i], batch2,
                params["Wout"], params["bout"].reshape(1, OUT),
            )
    return (graph, h)


# speedup vs baseline: 1.3916x; 1.3916x over previous
import functools

import jax
import jax.numpy as jnp
import numpy as np
from jax import lax
from jax.experimental import pallas as pl
from jax.experimental.pallas import tpu as pltpu
from jax.experimental.pallas import tpu_sc as plsc

N = 10000
E = 320000
D = 128
H = 128
OUT = 256
L = 5
B = 64
BN_EPS = 1e-5

NC = 2
NS = 16
NW = NC * NS
CHUNK = 128
NBUF = 2
EPW = E // NW
EPW_CH = -(-EPW // CHUNK)
PCH = 40
EPW_PAD = EPW_CH * CHUNK - EPW
NPAD = 10240
RPS = NPAD // NS

R = 1000
G = N // R


def _sc_scatter_add(h, src3, dst3, zeros_blk):
    mesh = plsc.VectorSubcoreMesh(core_axis_name="c", subcore_axis_name="s")

    @functools.partial(
        pl.kernel,
        mesh=mesh,
        out_type=jax.ShapeDtypeStruct((NC, NPAD, D), jnp.float32),
        scratch_types=[
            pltpu.VMEM((PCH, CHUNK), jnp.int32),
            pltpu.VMEM((PCH, CHUNK), jnp.int32),
            pltpu.VMEM((NBUF * CHUNK, D), jnp.float32),
            pltpu.VMEM_SHARED((NPAD, D), jnp.float32),
            pltpu.SemaphoreType.DMA,
            pltpu.SemaphoreType.DMA,
            pltpu.SemaphoreType.DMA,
            pltpu.SemaphoreType.DMA,
        ],
    )
    def k(
        h_hbm, src_hbm, dst_hbm, z_hbm, out_hbm,
        src_v, dst_v, rows_v, agg_sh, sga, sgb, ssa, ssb,
    ):
        c = lax.axis_index("c")
        s = lax.axis_index("s")
        wid = s * NC + c
        ra = rows_v.at[pl.ds(0, CHUNK)]
        rb = rows_v.at[pl.ds(CHUNK, CHUNK)]

        pltpu.sync_copy(z_hbm, ra)

        def zc(q, carry):
            pltpu.sync_copy(ra, agg_sh.at[pl.ds(s * RPS + q * CHUNK, CHUNK)])
            return carry

        lax.fori_loop(0, RPS // CHUNK, zc, 0)
        plsc.subcore_barrier()

        def drain(sem, buf):
            pltpu.make_async_copy(z_hbm, buf, sem).wait()

        for ph in range(2):
            npairs = PCH // 2 if ph == 0 else (EPW_CH - PCH) // 2
            base = 0 if ph == 0 else PCH
            nch = PCH if ph == 0 else EPW_CH - base
            pltpu.sync_copy(src_hbm.at[wid, pl.ds(base, nch)], src_v.at[pl.ds(0, nch)])
            pltpu.sync_copy(dst_hbm.at[wid, pl.ds(base, nch)], dst_v.at[pl.ds(0, nch)])

            def edge_pair(t, carry):
                j = t * 2
                pltpu.async_copy(h_hbm.at[src_v.at[j]], ra, sga)
                pltpu.async_copy(h_hbm.at[src_v.at[j + 1]], rb, sgb)
                drain(sga, ra)
                pltpu.async_copy(ra, agg_sh.at[dst_v.at[j]], ssa, add=True)
                drain(sgb, rb)
                pltpu.async_copy(rb, agg_sh.at[dst_v.at[j + 1]], ssb, add=True)
                drain(ssa, ra)
                drain(ssb, rb)
                return carry

            lax.fori_loop(0, npairs, edge_pair, 0)
            if nch % 2 == 1:
                jt = nch - 1
                pltpu.async_copy(h_hbm.at[src_v.at[jt]], ra, sga)
                drain(sga, ra)
                pltpu.async_copy(ra, agg_sh.at[dst_v.at[jt]], ssa, add=True)
                drain(ssa, ra)

        plsc.subcore_barrier()

        pltpu.sync_copy(
            agg_sh.at[pl.ds(s * RPS, RPS)], out_hbm.at[c, pl.ds(s * RPS, RPS)]
        )

    return k(h, src3, dst3, zeros_blk)


def _mlp_mid(h, aggs, W1, b1, W2, b2):
    def body(h_ref, a_ref, w1_ref, b1_ref, w2_ref, b2_ref, out_ref):
        z = h_ref[...] + a_ref[0] + a_ref[1]
        z = jnp.dot(z, w1_ref[...], preferred_element_type=jnp.float32) + b1_ref[...]
        z = jnp.maximum(z, 0.0)
        z = jnp.dot(z, w2_ref[...], preferred_element_type=jnp.float32) + b2_ref[...]
        z = jnp.maximum(z, 0.0)
        out_ref[...] = h_ref[...] + z

    return pl.pallas_call(
        body,
        grid=(G,),
        in_specs=[
            pl.BlockSpec((R, D), lambda i: (i, 0)),
            pl.BlockSpec((NC, R, D), lambda i: (0, i, 0)),
            pl.BlockSpec((D, H), lambda i: (0, 0)),
            pl.BlockSpec((1, H), lambda i: (0, 0)),
            pl.BlockSpec((H, H), lambda i: (0, 0)),
            pl.BlockSpec((1, H), lambda i: (0, 0)),
        ],
        out_specs=pl.BlockSpec((R, D), lambda i: (i, 0)),
        out_shape=jax.ShapeDtypeStruct((N, D), jnp.float32),
    )(h, aggs, W1, b1, W2, b2)


def _mlp_last(h, aggs, W1, b1, W2, b2, batch2, Wout, bout):
    def body(
        h_ref, a_ref, w1_ref, b1_ref, w2_ref, b2_ref, bt_ref, wo_ref, bo_ref,
        out_ref, g_ref, sums_ref, cnts_ref,
    ):
        i = pl.program_id(0)
        z = h_ref[...] + a_ref[0] + a_ref[1]
        z = jnp.dot(z, w1_ref[...], preferred_element_type=jnp.float32) + b1_ref[...]
        z = jnp.maximum(z, 0.0)
        z = jnp.dot(z, w2_ref[...], preferred_element_type=jnp.float32) + b2_ref[...]
        z = jnp.maximum(z, 0.0)
        hnew = h_ref[...] + z
        out_ref[...] = hnew

        onehot = (
            bt_ref[...] == lax.broadcasted_iota(jnp.int32, (R, B), 1)
        ).astype(jnp.float32)
        part = lax.dot_general(
            onehot, hnew, (((0,), (0,)), ((), ())),
            preferred_element_type=jnp.float32,
        )
        cnt = lax.dot_general(
            onehot, jnp.ones((R, 1), jnp.float32), (((0,), (0,)), ((), ())),
            preferred_element_type=jnp.float32,
        )

        @pl.when(i == 0)
        def _():
            sums_ref[...] = part
            cnts_ref[...] = cnt

        @pl.when(i > 0)
        def _():
            sums_ref[...] += part
            cnts_ref[...] += cnt

        @pl.when(i == G - 1)
        def _():
            mean = sums_ref[...] / jnp.maximum(cnts_ref[...], 1.0)
            g_ref[...] = (
                jnp.dot(mean, wo_ref[...], preferred_element_type=jnp.float32)
                + bo_ref[...]
            )

    return pl.pallas_call(
        body,
        grid=(G,),
        in_specs=[
            pl.BlockSpec((R, D), lambda i: (i, 0)),
            pl.BlockSpec((NC, R, D), lambda i: (0, i, 0)),
            pl.BlockSpec((D, H), lambda i: (0, 0)),
            pl.BlockSpec((1, H), lambda i: (0, 0)),
            pl.BlockSpec((H, H), lambda i: (0, 0)),
            pl.BlockSpec((1, H), lambda i: (0, 0)),
            pl.BlockSpec((R, 1), lambda i: (i, 0)),
            pl.BlockSpec((H, OUT), lambda i: (0, 0)),
            pl.BlockSpec((1, OUT), lambda i: (0, 0)),
        ],
        out_specs=[
            pl.BlockSpec((R, D), lambda i: (i, 0)),
            pl.BlockSpec((B, OUT), lambda i: (0, 0)),
        ],
        out_shape=[
            jax.ShapeDtypeStruct((N, D), jnp.float32),
            jax.ShapeDtypeStruct((B, OUT), jnp.float32),
        ],
        scratch_shapes=[
            pltpu.VMEM((B, H), jnp.float32),
            pltpu.VMEM((B, 1), jnp.float32),
        ],
    )(h, aggs, W1, b1, W2, b2, batch2, Wout, bout)


def kernel(x, edge_index, batch, params):
    inv = np.float32(1.0 / np.sqrt(1.0 + BN_EPS))

    src = edge_index[0].reshape(NW, EPW)
    dst = edge_index[1].reshape(NW, EPW)
    src3 = jnp.concatenate(
        [src, jnp.zeros((NW, EPW_PAD), jnp.int32)], axis=1
    ).reshape(NW, EPW_CH, CHUNK)
    dpad = N + (jnp.arange(EPW_PAD, dtype=jnp.int32) % (NPAD - N))
    dst3 = jnp.concatenate(
        [dst, jnp.broadcast_to(dpad[None, :], (NW, EPW_PAD))], axis=1
    ).reshape(NW, EPW_CH, CHUNK)
    zeros_blk = jnp.zeros((CHUNK, D), jnp.float32)
    batch2 = batch.reshape(N, 1)

    Ws1, bs1, Ws2, bs2 = [], [], [], []
    for i in range(L):
        g1 = params["l%d_g1" % i] * inv
        Ws1.append(params["l%d_W1" % i] * g1[None, :])
        bs1.append((params["l%d_b1" % i] * g1 + params["l%d_be1" % i]).reshape(1, H))
        g2 = params["l%d_bng" % i] * inv
        Ws2.append(params["l%d_W2" % i] * g2[None, :])
        bs2.append((params["l%d_b2" % i] * g2 + params["l%d_bnb" % i]).reshape(1, H))

    h = x
    for i in range(L):
        aggs = _sc_scatter_add(h, src3, dst3, zeros_blk)
        if i < L - 1:
            h = _mlp_mid(h, aggs, Ws1[i], bs1[i], Ws2[i], bs2[i])
        else:
            h, graph = _mlp_last(
                h, aggs, Ws1[i], bs1[i], Ws2[i], bs2[i], batch2,
                params["Wout"], params["bout"].reshape(1, OUT),
            )
    return (graph, h)


# speedup vs baseline: 1.4000x; 1.0060x over previous
import functools

import jax
import jax.numpy as jnp
import numpy as np
from jax import lax
from jax.experimental import pallas as pl
from jax.experimental.pallas import tpu as pltpu
from jax.experimental.pallas import tpu_sc as plsc

N = 10000
E = 320000
D = 128
H = 128
OUT = 256
L = 5
B = 64
BN_EPS = 1e-5

NC = 2
NS = 16
NW = NC * NS
CHUNK = 128
NBUF = 2
EPW = E // NW
EPW_CH = -(-EPW // CHUNK)
PCH = 40
EPW_PAD = EPW_CH * CHUNK - EPW
NPAD = 10240
RPS = NPAD // NS

R = 1000
G = N // R


def _sc_scatter_add(h, src3, dst3, zeros_blk):
    mesh = plsc.VectorSubcoreMesh(core_axis_name="c", subcore_axis_name="s")

    @functools.partial(
        pl.kernel,
        mesh=mesh,
        out_type=jax.ShapeDtypeStruct((NC, NPAD, D), jnp.float32),
        scratch_types=[
            pltpu.VMEM((PCH, CHUNK), jnp.int32),
            pltpu.VMEM((PCH, CHUNK), jnp.int32),
            pltpu.VMEM((NBUF * CHUNK, D), jnp.float32),
            pltpu.VMEM_SHARED((NPAD, D), jnp.float32),
            pltpu.SemaphoreType.DMA,
            pltpu.SemaphoreType.DMA,
            pltpu.SemaphoreType.DMA,
            pltpu.SemaphoreType.DMA,
        ],
    )
    def k(
        h_hbm, src_hbm, dst_hbm, z_hbm, out_hbm,
        src_v, dst_v, rows_v, agg_sh, sga, sgb, ssa, ssb,
    ):
        c = lax.axis_index("c")
        s = lax.axis_index("s")
        wid = s * NC + c
        ra = rows_v.at[pl.ds(0, CHUNK)]
        rb = rows_v.at[pl.ds(CHUNK, CHUNK)]

        def drain(sem, buf):
            pltpu.make_async_copy(z_hbm, buf, sem).wait()

        pltpu.sync_copy(z_hbm, ra)

        def zc(q, carry):
            pltpu.async_copy(ra, agg_sh.at[pl.ds(s * RPS + q * CHUNK, CHUNK)], ssa)
            return carry

        lax.fori_loop(0, RPS // CHUNK, zc, 0)

        for ph in range(2):
            npairs = PCH // 2 if ph == 0 else (EPW_CH - PCH) // 2
            base = 0 if ph == 0 else PCH
            nch = PCH if ph == 0 else EPW_CH - base
            pltpu.sync_copy(src_hbm.at[wid, pl.ds(base, nch)], src_v.at[pl.ds(0, nch)])
            pltpu.sync_copy(dst_hbm.at[wid, pl.ds(base, nch)], dst_v.at[pl.ds(0, nch)])

            if ph == 0:
                def zdrain(q, carry):
                    drain(ssa, ra)
                    return carry

                lax.fori_loop(0, RPS // CHUNK, zdrain, 0)
                plsc.subcore_barrier()

            def edge_pair(t, carry):
                j = t * 2
                pltpu.async_copy(h_hbm.at[src_v.at[j]], ra, sga)
                pltpu.async_copy(h_hbm.at[src_v.at[j + 1]], rb, sgb)
                drain(sga, ra)
                pltpu.async_copy(ra, agg_sh.at[dst_v.at[j]], ssa, add=True)
                drain(sgb, rb)
                pltpu.async_copy(rb, agg_sh.at[dst_v.at[j + 1]], ssb, add=True)
                drain(ssa, ra)
                drain(ssb, rb)
                return carry

            lax.fori_loop(0, npairs, edge_pair, 0)
            if nch % 2 == 1:
                jt = nch - 1
                pltpu.async_copy(h_hbm.at[src_v.at[jt]], ra, sga)
                drain(sga, ra)
                pltpu.async_copy(ra, agg_sh.at[dst_v.at[jt]], ssa, add=True)
                drain(ssa, ra)

        plsc.subcore_barrier()

        pltpu.sync_copy(
            agg_sh.at[pl.ds(s * RPS, RPS)], out_hbm.at[c, pl.ds(s * RPS, RPS)]
        )

    return k(h, src3, dst3, zeros_blk)


def _mlp_mid(h, aggs, W1, b1, W2, b2):
    def body(h_ref, a_ref, w1_ref, b1_ref, w2_ref, b2_ref, out_ref):
        z = h_ref[...] + a_ref[0] + a_ref[1]
        z = jnp.dot(z, w1_ref[...], preferred_element_type=jnp.float32) + b1_ref[...]
        z = jnp.maximum(z, 0.0)
        z = jnp.dot(z, w2_ref[...], preferred_element_type=jnp.float32) + b2_ref[...]
        z = jnp.maximum(z, 0.0)
        out_ref[...] = h_ref[...] + z

    return pl.pallas_call(
        body,
        grid=(G,),
        in_specs=[
            pl.BlockSpec((R, D), lambda i: (i, 0)),
            pl.BlockSpec((NC, R, D), lambda i: (0, i, 0)),
            pl.BlockSpec((D, H), lambda i: (0, 0)),
            pl.BlockSpec((1, H), lambda i: (0, 0)),
            pl.BlockSpec((H, H), lambda i: (0, 0)),
            pl.BlockSpec((1, H), lambda i: (0, 0)),
        ],
        out_specs=pl.BlockSpec((R, D), lambda i: (i, 0)),
        out_shape=jax.ShapeDtypeStruct((N, D), jnp.float32),
    )(h, aggs, W1, b1, W2, b2)


def _mlp_last(h, aggs, W1, b1, W2, b2, batch2, Wout, bout):
    def body(
        h_ref, a_ref, w1_ref, b1_ref, w2_ref, b2_ref, bt_ref, wo_ref, bo_ref,
        out_ref, g_ref, sums_ref, cnts_ref,
    ):
        i = pl.program_id(0)
        z = h_ref[...] + a_ref[0] + a_ref[1]
        z = jnp.dot(z, w1_ref[...], preferred_element_type=jnp.float32) + b1_ref[...]
        z = jnp.maximum(z, 0.0)
        z = jnp.dot(z, w2_ref[...], preferred_element_type=jnp.float32) + b2_ref[...]
        z = jnp.maximum(z, 0.0)
        hnew = h_ref[...] + z
        out_ref[...] = hnew

        onehot = (
            bt_ref[...] == lax.broadcasted_iota(jnp.int32, (R, B), 1)
        ).astype(jnp.float32)
        part = lax.dot_general(
            onehot, hnew, (((0,), (0,)), ((), ())),
            preferred_element_type=jnp.float32,
        )
        cnt = lax.dot_general(
            onehot, jnp.ones((R, 1), jnp.float32), (((0,), (0,)), ((), ())),
            preferred_element_type=jnp.float32,
        )

        @pl.when(i == 0)
        def _():
            sums_ref[...] = part
            cnts_ref[...] = cnt

        @pl.when(i > 0)
        def _():
            sums_ref[...] += part
            cnts_ref[...] += cnt

        @pl.when(i == G - 1)
        def _():
            mean = sums_ref[...] / jnp.maximum(cnts_ref[...], 1.0)
            g_ref[...] = (
                jnp.dot(mean, wo_ref[...], preferred_element_type=jnp.float32)
                + bo_ref[...]
            )

    return pl.pallas_call(
        body,
        grid=(G,),
        in_specs=[
            pl.BlockSpec((R, D), lambda i: (i, 0)),
            pl.BlockSpec((NC, R, D), lambda i: (0, i, 0)),
            pl.BlockSpec((D, H), lambda i: (0, 0)),
            pl.BlockSpec((1, H), lambda i: (0, 0)),
            pl.BlockSpec((H, H), lambda i: (0, 0)),
            pl.BlockSpec((1, H), lambda i: (0, 0)),
            pl.BlockSpec((R, 1), lambda i: (i, 0)),
            pl.BlockSpec((H, OUT), lambda i: (0, 0)),
            pl.BlockSpec((1, OUT), lambda i: (0, 0)),
        ],
        out_specs=[
            pl.BlockSpec((R, D), lambda i: (i, 0)),
            pl.BlockSpec((B, OUT), lambda i: (0, 0)),
        ],
        out_shape=[
            jax.ShapeDtypeStruct((N, D), jnp.float32),
            jax.ShapeDtypeStruct((B, OUT), jnp.float32),
        ],
        scratch_shapes=[
            pltpu.VMEM((B, H), jnp.float32),
            pltpu.VMEM((B, 1), jnp.float32),
        ],
    )(h, aggs, W1, b1, W2, b2, batch2, Wout, bout)


def kernel(x, edge_index, batch, params):
    inv = np.float32(1.0 / np.sqrt(1.0 + BN_EPS))

    src = edge_index[0].reshape(NW, EPW)
    dst = edge_index[1].reshape(NW, EPW)
    src3 = jnp.concatenate(
        [src, jnp.zeros((NW, EPW_PAD), jnp.int32)], axis=1
    ).reshape(NW, EPW_CH, CHUNK)
    dpad = N + (jnp.arange(EPW_PAD, dtype=jnp.int32) % (NPAD - N))
    dst3 = jnp.concatenate(
        [dst, jnp.broadcast_to(dpad[None, :], (NW, EPW_PAD))], axis=1
    ).reshape(NW, EPW_CH, CHUNK)
    zeros_blk = jnp.zeros((CHUNK, D), jnp.float32)
    batch2 = batch.reshape(N, 1)

    Ws1, bs1, Ws2, bs2 = [], [], [], []
    for i in range(L):
        g1 = params["l%d_g1" % i] * inv
        Ws1.append(params["l%d_W1" % i] * g1[None, :])
        bs1.append((params["l%d_b1" % i] * g1 + params["l%d_be1" % i]).reshape(1, H))
        g2 = params["l%d_bng" % i] * inv
        Ws2.append(params["l%d_W2" % i] * g2[None, :])
        bs2.append((params["l%d_b2" % i] * g2 + params["l%d_bnb" % i]).reshape(1, H))

    h = x
    for i in range(L):
        aggs = _sc_scatter_add(h, src3, dst3, zeros_blk)
        if i < L - 1:
            h = _mlp_mid(h, aggs, Ws1[i], bs1[i], Ws2[i], bs2[i])
        else:
            h, graph = _mlp_last(
                h, aggs, Ws1[i], bs1[i], Ws2[i], bs2[i], batch2,
                params["Wout"], params["bout"].reshape(1, OUT),
            )
    return (graph, h)
